# baseline (speedup 1.0000x reference)
import jax
import jax.numpy as jnp
from jax.experimental import pallas as pl

_K = 512
_D = 64
_HW = 1024
_BB = 2


def _vq_kernel(x_ref, cb_ref, enc_ref, q_ref):
    cb = cb_ref[...]
    c_sq = jnp.sum(cb * cb, axis=1, keepdims=True)
    for s in range(_BB):
        xb = x_ref[s]
        scores = jax.lax.dot_general(
            cb, xb, (((1,), (0,)), ((), ())),
            preferred_element_type=jnp.float32)
        x_sq = jnp.sum(xb * xb, axis=0, keepdims=True)
        d2 = x_sq - 2.0 * scores + c_sq
        dist = jnp.sqrt(jnp.maximum(d2, 0.0))
        idx = jnp.argmin(dist, axis=0)
        enc = (jax.lax.broadcasted_iota(jnp.int32, (_HW, _K), 1)
               == idx[:, None]).astype(jnp.float32)
        enc_ref[pl.ds(s * _HW, _HW), :] = enc
        q = jax.lax.dot_general(
            cb, enc, (((0,), (1,)), ((), ())),
            preferred_element_type=jnp.float32)
        q_ref[s] = q


def kernel(x, codebook):
    b = x.shape[0]
    n = b * _HW
    x3 = x.reshape(b, _D, _HW)
    enc, q = pl.pallas_call(
        _vq_kernel,
        grid=(b // _BB,),
        in_specs=[
            pl.BlockSpec((_BB, _D, _HW), lambda i: (i, 0, 0)),
            pl.BlockSpec((_K, _D), lambda i: (0, 0)),
        ],
        out_specs=[
            pl.BlockSpec((_BB * _HW, _K), lambda i: (i, 0)),
            pl.BlockSpec((_BB, _D, _HW), lambda i: (i, 0, 0)),
        ],
        out_shape=[
            jax.ShapeDtypeStruct((n, _K), jnp.float32),
            jax.ShapeDtypeStruct((b, _D, _HW), jnp.float32),
        ],
    )(x3, codebook)
    return (enc, q.reshape(x.shape))
